# Initial kernel scaffold; baseline (speedup 1.0000x reference)
import jax
import jax.numpy as jnp
from jax.experimental import pallas as pl


def kernel(x, token_table, pos_table):
    raise NotImplementedError("write your pallas kernel here")


# speedup vs baseline: 3.6965x; 3.6965x over previous
import jax
import jax.numpy as jnp
from jax import lax
from jax.experimental import pallas as pl
from jax.experimental.pallas import tpu as pltpu
from jax.experimental.pallas import tpu_sc as plsc

_MAXLEN = 200
_EMBED = 64
_BATCH = 4096
_LANES = 16

_NC = 2
_NS = 16
_NW = _NC * _NS
_BPW = _BATCH // _NW
_G = 4
_ROWS = _G * _MAXLEN
_GATHER = 80
_NG = _ROWS // _GATHER
_CHUNKS = _BPW // _G
_J = _EMBED // _LANES


def _tec_body(x_hbm, tok_hbm, pos_hbm, out_hbm, pos_v, idx_v, rows_v, gsem):
    c = lax.axis_index("c")
    s = lax.axis_index("s")
    wid = s * _NC + c
    pltpu.sync_copy(pos_hbm, pos_v)
    row_base = wid * _BPW * _MAXLEN

    def chunk_body(i, carry):
        row0 = row_base + i * _ROWS
        pltpu.sync_copy(x_hbm.at[pl.ds(row0, _ROWS)], idx_v)
        copies = [
            pltpu.async_copy(
                tok_hbm.at[idx_v.at[pl.ds(k * _GATHER, _GATHER)]],
                rows_v.at[pl.ds(k * _GATHER, _GATHER)],
                gsem,
            )
            for k in range(_NG)
        ]
        for cp in copies:
            cp.wait()

        def t_body(t, c2):
            for j in range(_J):
                p = pos_v[t, pl.ds(j * _LANES, _LANES)]
                for b in range(_G):
                    r = b * _MAXLEN + t
                    rows_v[r, pl.ds(j * _LANES, _LANES)] = (
                        rows_v[r, pl.ds(j * _LANES, _LANES)] + p
                    )
            return c2

        lax.fori_loop(0, _MAXLEN, t_body, 0)
        pltpu.sync_copy(rows_v, out_hbm.at[pl.ds(row0, _ROWS)])
        return carry

    lax.fori_loop(0, _CHUNKS, chunk_body, 0)


def kernel(x, token_table, pos_table):
    x_flat = x.reshape(-1).astype(jnp.int32)
    mesh = plsc.VectorSubcoreMesh(core_axis_name="c", subcore_axis_name="s")
    out = pl.kernel(
        _tec_body,
        out_type=jax.ShapeDtypeStruct((_BATCH * _MAXLEN, _EMBED), jnp.float32),
        mesh=mesh,
        compiler_params=pltpu.CompilerParams(use_tc_tiling_on_sc=False),
        scratch_types=[
            pltpu.VMEM((_MAXLEN, _EMBED), jnp.float32),
            pltpu.VMEM((_ROWS,), jnp.int32),
            pltpu.VMEM((_ROWS, _EMBED), jnp.float32),
            pltpu.SemaphoreType.DMA,
        ],
    )(x_flat, token_table, pos_table)
    return out.reshape(_BATCH, _MAXLEN, _EMBED)


# speedup vs baseline: 4.0903x; 1.1065x over previous
import jax
import jax.numpy as jnp
from jax import lax
from jax.experimental import pallas as pl
from jax.experimental.pallas import tpu as pltpu
from jax.experimental.pallas import tpu_sc as plsc

_MAXLEN = 200
_EMBED = 64
_BATCH = 4096
_LANES = 16

_NC = 2
_NS = 16
_NW = _NC * _NS
_BPW = _BATCH // _NW
_G = 4
_ROWS = _G * _MAXLEN
_GATHER = 80
_NG = _ROWS // _GATHER
_CHUNKS = _BPW // _G
_J = _EMBED // _LANES


def _tec_body(x_hbm, tok_hbm, pos_hbm, out_hbm, pos_v, idx_v, rows_v,
              gsem0, gsem1, osem0, osem1):
    c = lax.axis_index("c")
    s = lax.axis_index("s")
    wid = s * _NC + c
    gsems = (gsem0, gsem1)
    osems = (osem0, osem1)
    pltpu.sync_copy(pos_hbm, pos_v)
    row_base = wid * _BPW * _MAXLEN

    def add_positions(slot):
        @plsc.parallel_loop(0, _MAXLEN, unroll=2)
        def _(t):
            for j in range(_J):
                p = pos_v[t, pl.ds(j * _LANES, _LANES)]
                for b in range(_G):
                    r = b * _MAXLEN + t
                    rows_v[slot, r, pl.ds(j * _LANES, _LANES)] = (
                        rows_v[slot, r, pl.ds(j * _LANES, _LANES)] + p
                    )

    gds = [None] * _CHUNKS
    ods = [None] * _CHUNKS
    for i in range(_CHUNKS + 1):
        if i < _CHUNKS:
            slot = i % 2
            if i >= 2:
                ods[i - 2].wait()
            row0 = row_base + i * _ROWS
            pltpu.sync_copy(x_hbm.at[pl.ds(row0, _ROWS)], idx_v.at[slot])
            gds[i] = [
                pltpu.async_copy(
                    tok_hbm.at[idx_v.at[slot, pl.ds(k * _GATHER, _GATHER)]],
                    rows_v.at[slot, pl.ds(k * _GATHER, _GATHER)],
                    gsems[slot],
                )
                for k in range(_NG)
            ]
        if i >= 1:
            k = i - 1
            slot = k % 2
            for d in gds[k]:
                d.wait()
            add_positions(slot)
            ods[k] = pltpu.async_copy(
                rows_v.at[slot],
                out_hbm.at[pl.ds(row_base + k * _ROWS, _ROWS)],
                osems[slot],
            )
    ods[_CHUNKS - 2].wait()
    ods[_CHUNKS - 1].wait()


def kernel(x, token_table, pos_table):
    x_flat = x.reshape(-1).astype(jnp.int32)
    mesh = plsc.VectorSubcoreMesh(core_axis_name="c", subcore_axis_name="s")
    out = pl.kernel(
        _tec_body,
        out_type=jax.ShapeDtypeStruct((_BATCH * _MAXLEN, _EMBED), jnp.float32),
        mesh=mesh,
        compiler_params=pltpu.CompilerParams(use_tc_tiling_on_sc=False),
        scratch_types=[
            pltpu.VMEM((_MAXLEN, _EMBED), jnp.float32),
            pltpu.VMEM((2, _ROWS), jnp.int32),
            pltpu.VMEM((2, _ROWS, _EMBED), jnp.float32),
            pltpu.SemaphoreType.DMA,
            pltpu.SemaphoreType.DMA,
            pltpu.SemaphoreType.DMA,
            pltpu.SemaphoreType.DMA,
        ],
    )(x_flat, token_table, pos_table)
    return out.reshape(_BATCH, _MAXLEN, _EMBED)
